# baseline (speedup 1.0000x reference)
import jax
import jax.numpy as jnp
from jax.experimental import pallas as pl
from jax.experimental.pallas import tpu as pltpu

_B, _S, _D = 4, 8192, 768
_RANK = 4
_E = 16
_SCALE = 1.0 / _RANK

_TS = 4096


def _fused_kernel(sid_ref, x_ref, Wt_ref, b_ref, A_ref, B_ref, out_ref, weff_ref):
    @pl.when(pl.program_id(1) == 0)
    def _build_weff():
        upd = jax.lax.dot_general(
            A_ref[0],
            B_ref[0],
            (((0,), (1,)), ((), ())),
            preferred_element_type=jnp.float32,
        )
        weff_ref[...] = Wt_ref[...] + _SCALE * upd

    out_ref[0] = (
        jnp.dot(x_ref[0], weff_ref[...], preferred_element_type=jnp.float32)
        + b_ref[...]
    )


def kernel(x, subject_id, W, b, lora_A, lora_B):
    Wt = W.T
    sid = subject_id.astype(jnp.int32)
    n_s = _S // _TS

    grid_spec = pltpu.PrefetchScalarGridSpec(
        num_scalar_prefetch=1,
        grid=(_B, n_s),
        in_specs=[
            pl.BlockSpec((1, _TS, _D), lambda bb, ss, sid_ref: (bb, ss, 0)),
            pl.BlockSpec((_D, _D), lambda bb, ss, sid_ref: (0, 0)),
            pl.BlockSpec((1, _D), lambda bb, ss, sid_ref: (0, 0)),
            pl.BlockSpec((1, _RANK, _D), lambda bb, ss, sid_ref: (sid_ref[bb], 0, 0)),
            pl.BlockSpec((1, _D, _RANK), lambda bb, ss, sid_ref: (sid_ref[bb], 0, 0)),
        ],
        out_specs=pl.BlockSpec((1, _TS, _D), lambda bb, ss, sid_ref: (bb, ss, 0)),
        scratch_shapes=[pltpu.VMEM((_D, _D), jnp.float32)],
    )

    return pl.pallas_call(
        _fused_kernel,
        grid_spec=grid_spec,
        out_shape=jax.ShapeDtypeStruct((_B, _S, _D), jnp.float32),
        compiler_params=pltpu.CompilerParams(
            dimension_semantics=("arbitrary", "arbitrary"),
            vmem_limit_bytes=100 * 1024 * 1024,
        ),
    )(sid, x, Wt, b.reshape(1, _D), lora_A, lora_B)


# speedup vs baseline: 1.0421x; 1.0421x over previous
import jax
import jax.numpy as jnp
from jax.experimental import pallas as pl
from jax.experimental.pallas import tpu as pltpu

_B, _S, _D = 4, 8192, 768
_RANK = 4
_E = 16
_SCALE = 1.0 / _RANK

_TS = 4096


def _fused_kernel(sid_ref, x_ref, Wt_ref, b_ref, A_ref, B_ref, out_ref, weff_ref):
    @pl.when(pl.program_id(1) == 0)
    def _build_weff():
        weff_ref[...] = Wt_ref[...] + _SCALE * jnp.dot(
            B_ref[0], A_ref[0], preferred_element_type=jnp.float32
        )

    out_ref[0] = (
        jax.lax.dot_general(
            x_ref[0],
            weff_ref[...],
            (((1,), (1,)), ((), ())),
            preferred_element_type=jnp.float32,
        )
        + b_ref[...]
    )


def kernel(x, subject_id, W, b, lora_A, lora_B):
    sid = subject_id.astype(jnp.int32)
    n_s = _S // _TS

    grid_spec = pltpu.PrefetchScalarGridSpec(
        num_scalar_prefetch=1,
        grid=(_B, n_s),
        in_specs=[
            pl.BlockSpec((1, _TS, _D), lambda bb, ss, sid_ref: (bb, ss, 0)),
            pl.BlockSpec((_D, _D), lambda bb, ss, sid_ref: (0, 0)),
            pl.BlockSpec((1, _D), lambda bb, ss, sid_ref: (0, 0)),
            pl.BlockSpec((1, _RANK, _D), lambda bb, ss, sid_ref: (sid_ref[bb], 0, 0)),
            pl.BlockSpec((1, _D, _RANK), lambda bb, ss, sid_ref: (sid_ref[bb], 0, 0)),
        ],
        out_specs=pl.BlockSpec((1, _TS, _D), lambda bb, ss, sid_ref: (bb, ss, 0)),
        scratch_shapes=[pltpu.VMEM((_D, _D), jnp.float32)],
    )

    return pl.pallas_call(
        _fused_kernel,
        grid_spec=grid_spec,
        out_shape=jax.ShapeDtypeStruct((_B, _S, _D), jnp.float32),
        compiler_params=pltpu.CompilerParams(
            dimension_semantics=("arbitrary", "arbitrary"),
            vmem_limit_bytes=100 * 1024 * 1024,
        ),
    )(sid, x, W, b.reshape(1, _D), lora_A, lora_B)


# speedup vs baseline: 1.0690x; 1.0258x over previous
import jax
import jax.numpy as jnp
from jax.experimental import pallas as pl
from jax.experimental.pallas import tpu as pltpu

_B, _S, _D = 4, 8192, 768
_RANK = 4
_E = 16
_SCALE = 1.0 / _RANK

_TS = 4096


def _fused_kernel(sid_ref, x_ref, Wt_ref, b_ref, A_ref, Bt_ref, out_ref, weff_ref):
    @pl.when(pl.program_id(1) == 0)
    def _build_weff():
        weff_ref[...] = (
            Wt_ref[...]
            + _SCALE
            * jnp.dot(A_ref[0].T, Bt_ref[0], preferred_element_type=jnp.float32)
        ).astype(jnp.bfloat16)

    out_ref[0] = (
        jnp.dot(
            x_ref[0].astype(jnp.bfloat16),
            weff_ref[...],
            preferred_element_type=jnp.float32,
        )
        + b_ref[...]
    )


def kernel(x, subject_id, W, b, lora_A, lora_B):
    Wt = W.T
    Bt = lora_B.transpose(0, 2, 1)
    sid = subject_id.astype(jnp.int32)
    n_s = _S // _TS

    grid_spec = pltpu.PrefetchScalarGridSpec(
        num_scalar_prefetch=1,
        grid=(_B, n_s),
        in_specs=[
            pl.BlockSpec((1, _TS, _D), lambda bb, ss, sid_ref: (bb, ss, 0)),
            pl.BlockSpec((_D, _D), lambda bb, ss, sid_ref: (0, 0)),
            pl.BlockSpec((1, _D), lambda bb, ss, sid_ref: (0, 0)),
            pl.BlockSpec((1, _RANK, _D), lambda bb, ss, sid_ref: (sid_ref[bb], 0, 0)),
            pl.BlockSpec((1, _RANK, _D), lambda bb, ss, sid_ref: (sid_ref[bb], 0, 0)),
        ],
        out_specs=pl.BlockSpec((1, _TS, _D), lambda bb, ss, sid_ref: (bb, ss, 0)),
        scratch_shapes=[pltpu.VMEM((_D, _D), jnp.bfloat16)],
    )

    return pl.pallas_call(
        _fused_kernel,
        grid_spec=grid_spec,
        out_shape=jax.ShapeDtypeStruct((_B, _S, _D), jnp.float32),
        compiler_params=pltpu.CompilerParams(
            dimension_semantics=("arbitrary", "arbitrary"),
            vmem_limit_bytes=124 * 1024 * 1024,
        ),
    )(sid, x, Wt, b.reshape(1, _D), lora_A, Bt)
